# baseline (speedup 1.0000x reference)
import functools

import jax
import jax.numpy as jnp
from jax import lax
from jax.experimental import pallas as pl
from jax.experimental.pallas import tpu as pltpu
from jax.experimental.pallas import tpu_sc as plsc

VOCAB = 1000
D = 4
DP = 16
NC = 2
NS = 16
NW = NC * NS
CHUNK = 128
RB = 1024


def _sc_gather(table_p, ids3):
    n_chunks = ids3.shape[1]
    mesh = plsc.VectorSubcoreMesh(core_axis_name="c", subcore_axis_name="s")

    @functools.partial(
        pl.kernel,
        mesh=mesh,
        out_type=jax.ShapeDtypeStruct((NW, n_chunks, CHUNK, DP), jnp.float32),
        scratch_types=[
            pltpu.VMEM((n_chunks, CHUNK), jnp.int32),
            pltpu.VMEM((n_chunks, CHUNK, DP), jnp.float32),
            pltpu.SemaphoreType.DMA,
        ],
        compiler_params=pltpu.CompilerParams(use_tc_tiling_on_sc=False),
    )
    def k(table_hbm, ids_hbm, out_hbm, idx_v, rows_v, sem):
        wid = lax.axis_index("s") * NC + lax.axis_index("c")
        pltpu.sync_copy(ids_hbm.at[wid], idx_v)

        def body(j, carry):
            pltpu.async_copy(table_hbm.at[idx_v.at[j]], rows_v.at[j], sem).wait()
            return carry

        lax.fori_loop(0, n_chunks, body, 0)
        pltpu.sync_copy(rows_v, out_hbm.at[wid])

    return k(table_p, ids3)


def _mm_body(h_ref, w_ref, o_ref):
    o_ref[...] = lax.dot_general(
        h_ref[...], w_ref[...],
        (((1,), (0,)), ((), ())),
        preferred_element_type=jnp.float32,
    )


def _tc_matmul(h_pad, head_t):
    btot = h_pad.shape[0]
    return pl.pallas_call(
        _mm_body,
        grid=(btot // RB,),
        in_specs=[
            pl.BlockSpec((RB, DP), lambda i: (i, 0)),
            pl.BlockSpec((DP, VOCAB), lambda i: (0, 0)),
        ],
        out_specs=pl.BlockSpec((RB, VOCAB), lambda i: (i, 0)),
        out_shape=jax.ShapeDtypeStruct((btot, VOCAB), jnp.float32),
    )(h_pad, head_t)


def kernel(input_ids, emb_weight, head_weight):
    b, t = input_ids.shape
    btot = b * t
    ids3 = input_ids.astype(jnp.int32).reshape(NW, btot // (NW * CHUNK), CHUNK)
    emb_p = jnp.pad(emb_weight, ((0, 0), (0, DP - D)))
    head_t = jnp.pad(head_weight, ((0, 0), (0, DP - D))).T
    h = _sc_gather(emb_p, ids3).reshape(btot, DP)
    logits = _tc_matmul(h, head_t)
    return logits.reshape(b, t, VOCAB)


# speedup vs baseline: 1.3406x; 1.3406x over previous
import functools

import jax
import jax.numpy as jnp
from jax import lax
from jax.experimental import pallas as pl
from jax.experimental.pallas import tpu as pltpu
from jax.experimental.pallas import tpu_sc as plsc

VOCAB = 1000
D = 4
DP = 16
NC = 2
NS = 16
NW = NC * NS
CHUNK = 128
BB = 32


def _sc_gather(table_p, ids3, btot):
    n_chunks = ids3.shape[1]
    per_w = n_chunks * CHUNK
    mesh = plsc.VectorSubcoreMesh(core_axis_name="c", subcore_axis_name="s")

    @functools.partial(
        pl.kernel,
        mesh=mesh,
        out_type=jax.ShapeDtypeStruct((btot, DP), jnp.float32),
        scratch_types=[
            pltpu.VMEM((n_chunks, CHUNK), jnp.int32),
            pltpu.VMEM((per_w, DP), jnp.float32),
            pltpu.SemaphoreType.DMA,
        ],
        compiler_params=pltpu.CompilerParams(use_tc_tiling_on_sc=False),
    )
    def k(table_hbm, ids_hbm, out_hbm, idx_v, rows_v, sem):
        wid = lax.axis_index("s") * NC + lax.axis_index("c")
        pltpu.sync_copy(ids_hbm.at[wid], idx_v)

        def body(j, carry):
            pltpu.async_copy(
                table_hbm.at[idx_v.at[j]], rows_v.at[pl.ds(j * CHUNK, CHUNK)], sem
            ).wait()
            return carry

        lax.fori_loop(0, n_chunks, body, 0)
        pltpu.sync_copy(rows_v, out_hbm.at[pl.ds(wid * per_w, per_w)])

    return k(table_p, ids3)


def _tc_matmul(h, head_t, b, t):
    def _mm_body(h_ref, w_ref, o_ref):
        w = w_ref[...]
        for bb in range(BB):
            o_ref[bb] = lax.dot_general(
                h_ref[pl.ds(bb * t, t), :], w,
                (((1,), (0,)), ((), ())),
                preferred_element_type=jnp.float32,
            )

    return pl.pallas_call(
        _mm_body,
        grid=(b // BB,),
        in_specs=[
            pl.BlockSpec((BB * t, DP), lambda i: (i, 0)),
            pl.BlockSpec((DP, VOCAB), lambda i: (0, 0)),
        ],
        out_specs=pl.BlockSpec((BB, t, VOCAB), lambda i: (i, 0, 0)),
        out_shape=jax.ShapeDtypeStruct((b, t, VOCAB), jnp.float32),
    )(h, head_t)


def kernel(input_ids, emb_weight, head_weight):
    b, t = input_ids.shape
    btot = b * t
    ids3 = input_ids.astype(jnp.int32).reshape(NW, btot // (NW * CHUNK), CHUNK)
    emb_p = jnp.pad(emb_weight, ((0, 0), (0, DP - D)))
    head_t = jnp.pad(head_weight, ((0, 0), (0, DP - D))).T
    h = _sc_gather(emb_p, ids3, btot)
    return _tc_matmul(h, head_t, b, t)


# speedup vs baseline: 1.5572x; 1.1616x over previous
import functools

import jax
import jax.numpy as jnp
from jax import lax
from jax.experimental import pallas as pl
from jax.experimental.pallas import tpu as pltpu
from jax.experimental.pallas import tpu_sc as plsc

VOCAB = 1000
D = 4
DP = 16
NC = 2
NS = 16
NW = NC * NS
CHUNK = 128
BB = 16
NBUF = 4


def _sc_gather(table_p, ids3, btot):
    n_chunks = ids3.shape[1]
    per_w = n_chunks * CHUNK
    mesh = plsc.VectorSubcoreMesh(core_axis_name="c", subcore_axis_name="s")

    @functools.partial(
        pl.kernel,
        mesh=mesh,
        out_type=jax.ShapeDtypeStruct((btot, DP), jnp.float32),
        scratch_types=[
            pltpu.VMEM((n_chunks, CHUNK), jnp.int32),
            pltpu.VMEM((per_w, DP), jnp.float32),
            pltpu.SemaphoreType.DMA,
        ],
        compiler_params=pltpu.CompilerParams(use_tc_tiling_on_sc=False),
    )
    def k(table_hbm, ids_hbm, out_hbm, idx_v, rows_v, sem):
        wid = lax.axis_index("s") * NC + lax.axis_index("c")
        pltpu.sync_copy(ids_hbm.at[wid], idx_v)

        def body(j, carry):
            pltpu.async_copy(
                table_hbm.at[idx_v.at[j]], rows_v.at[pl.ds(j * CHUNK, CHUNK)], sem
            ).wait()
            return carry

        lax.fori_loop(0, n_chunks, body, 0)
        pltpu.sync_copy(rows_v, out_hbm.at[pl.ds(wid * per_w, per_w)])

    return k(table_p, ids3)


TP = 56
VP = 1024


def _tc_matmul(h, head_t, b, t):
    grid_n = b // BB

    def _mm_body(h_ref, w_ref, o_ref):
        w = w_ref[...]
        for bb in range(BB):
            o_ref[bb, :t, :] = lax.dot_general(
                h_ref[pl.ds(bb * t, t), :], w,
                (((1,), (0,)), ((), ())),
                preferred_element_type=jnp.float32,
            )

    return pl.pallas_call(
        _mm_body,
        grid=(grid_n,),
        in_specs=[
            pl.BlockSpec((BB * t, DP), lambda i: (i, 0)),
            pl.BlockSpec((DP, VP), lambda i: (0, 0)),
        ],
        out_specs=pl.BlockSpec((BB, TP, VP), lambda i: (i, 0, 0)),
        out_shape=jax.ShapeDtypeStruct((b, TP, VP), jnp.float32),
    )(h, head_t)


def kernel(input_ids, emb_weight, head_weight):
    b, t = input_ids.shape
    btot = b * t
    ids3 = input_ids.astype(jnp.int32).reshape(NW, btot // (NW * CHUNK), CHUNK)
    emb_p = jnp.pad(emb_weight, ((0, 0), (0, DP - D)))
    head_t = jnp.pad(jnp.pad(head_weight, ((0, 0), (0, DP - D))).T,
                     ((0, 0), (0, VP - VOCAB)))
    h = _sc_gather(emb_p, ids3, btot)
    out_p = _tc_matmul(h, head_t, b, t)
    return out_p[:, :t, :VOCAB]
